# baseline (speedup 1.0000x reference)
import functools

import jax
import jax.numpy as jnp
from jax import lax
from jax.experimental import pallas as pl
from jax.experimental.pallas import tpu as pltpu
from jax.experimental.pallas import tpu_sc as plsc

_DIM = 512
_CBG = 32
_CBB = 32
_NG = 80
_NB = 480
_L = 16


@functools.cache
def _make_lookup(B, D, V):
    info = plsc.get_sparse_core_info()
    NC, NS = info.num_cores, info.num_subcores
    NW = NC * NS
    DQ = D // 4
    g_per_w = _NG * _CBG
    b_per_s = _NB * _CBB
    G_TOTAL = NW * g_per_w
    assert G_TOTAL + (NW // 4) * b_per_s == B
    NP = _NG // 2
    assert _NB // 12 == NP
    mesh = plsc.VectorSubcoreMesh(core_axis_name="c", subcore_axis_name="s")

    @functools.partial(
        pl.kernel,
        mesh=mesh,
        out_type=jax.ShapeDtypeStruct((B, D), jnp.float32),
        scratch_types=[
            pltpu.VMEM((g_per_w,), jnp.int32),
            pltpu.VMEM((b_per_s,), jnp.int32),
            pltpu.VMEM((V, DQ), jnp.float32),
            [pltpu.VMEM((_CBG, D), jnp.float32) for _ in range(2)],
            [pltpu.VMEM((_CBB, DQ), jnp.float32) for _ in range(2)],
            [pltpu.SemaphoreType.DMA for _ in range(2)],
            [pltpu.SemaphoreType.DMA for _ in range(2)],
            [pltpu.SemaphoreType.DMA for _ in range(2)],
        ],
    )
    def lookup(table_hbm, idx_hbm, out_hbm,
               idx_vg, idx_vb, tbl_v, gbuf, bbuf, sg, ssg, ssb):
        wid = lax.axis_index("s") * NC + lax.axis_index("c")
        gbase = wid * g_per_w
        span = wid // 4
        quarter = wid % 4
        bbase = G_TOTAL + span * b_per_s
        col = quarter * DQ
        pltpu.sync_copy(idx_hbm.at[pl.ds(gbase, g_per_w)], idx_vg)
        pltpu.sync_copy(idx_hbm.at[pl.ds(bbase, b_per_s)], idx_vb)
        pltpu.sync_copy(table_hbm.at[:, pl.ds(col, DQ)], tbl_v)

        def g_start(i, j):
            pltpu.async_copy(
                table_hbm.at[idx_vg.at[pl.ds(i * _CBG, _CBG)]], gbuf[j],
                sg[j])

        def g_wait(i, j):
            pltpu.make_async_copy(
                table_hbm.at[idx_vg.at[pl.ds(i * _CBG, _CBG)]], gbuf[j],
                sg[j]).wait()

        def g_out(i, j):
            pltpu.async_copy(
                gbuf[j], out_hbm.at[pl.ds(gbase + i * _CBG, _CBG)], ssg[j])

        def g_out_wait(i, j):
            pltpu.make_async_copy(
                gbuf[j], out_hbm.at[pl.ds(gbase + i * _CBG, _CBG)],
                ssg[j]).wait()

        def build(i, j):
            off = i * _CBB
            for g in range(_CBB // _L):
                vec = idx_vb[pl.ds(off + g * _L, _L)]
                rs = [vec[k] for k in range(_L)]

                @plsc.parallel_loop(0, DQ // _L, 1, unroll=DQ // _L)
                def col_body(jj):
                    for k in range(_L):
                        bbuf[j][g * _L + k, pl.ds(jj * _L, _L)] = (
                            tbl_v[rs[k], pl.ds(jj * _L, _L)])

        def b_out(i, j):
            pltpu.async_copy(
                bbuf[j],
                out_hbm.at[pl.ds(bbase + i * _CBB, _CBB), pl.ds(col, DQ)],
                ssb[j])

        def b_out_wait(i, j):
            pltpu.make_async_copy(
                bbuf[j],
                out_hbm.at[pl.ds(bbase + i * _CBB, _CBB), pl.ds(col, DQ)],
                ssb[j]).wait()

        def body(p, carry):
            for t in range(2):
                ig = 2 * p + t
                jg = t

                @pl.when(ig >= 1)
                def _():
                    g_wait(ig - 1, 1 - jg)
                    g_out(ig - 1, 1 - jg)

                @pl.when(ig >= 2)
                def _():
                    g_out_wait(ig - 2, jg)

                g_start(ig, jg)

                for u in range(6):
                    ib = 12 * p + 6 * t + u
                    jb = (6 * t + u) % 2

                    @pl.when(ib >= 2)
                    def _():
                        b_out_wait(ib - 2, jb)

                    build(ib, jb)
                    b_out(ib, jb)
            return carry

        lax.fori_loop(0, NP, body, 0)
        g_wait(_NG - 1, (_NG - 1) % 2)
        g_out(_NG - 1, (_NG - 1) % 2)
        g_out_wait(_NG - 2, (_NG - 2) % 2)
        g_out_wait(_NG - 1, (_NG - 1) % 2)
        b_out_wait(_NB - 2, 0)
        b_out_wait(_NB - 1, 1)

    return lookup


def kernel(ids, kernel):
    rows, cols = ids.shape
    B = rows * cols
    idx = ids.reshape(B).astype(jnp.int32)
    out = _make_lookup(B, _DIM, kernel.shape[0])(kernel, idx)
    return out.reshape(rows, cols, _DIM)


# speedup vs baseline: 1.0613x; 1.0613x over previous
import functools

import jax
import jax.numpy as jnp
from jax import lax
from jax.experimental import pallas as pl
from jax.experimental.pallas import tpu as pltpu
from jax.experimental.pallas import tpu_sc as plsc

_DIM = 512
_CBG = 24
_CBB = 32
_NG = 100
_NB = 500
_L = 16


@functools.cache
def _make_lookup(B, D, V):
    info = plsc.get_sparse_core_info()
    NC, NS = info.num_cores, info.num_subcores
    NW = NC * NS
    DQ = D // 4
    g_per_w = _NG * _CBG
    b_per_s = _NB * _CBB
    G_TOTAL = NW * g_per_w
    assert G_TOTAL + (NW // 4) * b_per_s == B
    NP = _NG // 4
    assert _NB // 20 == NP
    mesh = plsc.VectorSubcoreMesh(core_axis_name="c", subcore_axis_name="s")

    @functools.partial(
        pl.kernel,
        mesh=mesh,
        out_type=jax.ShapeDtypeStruct((B, D), jnp.float32),
        scratch_types=[
            pltpu.VMEM((g_per_w,), jnp.int32),
            pltpu.VMEM((b_per_s,), jnp.int32),
            pltpu.VMEM((V, DQ), jnp.float32),
            [pltpu.VMEM((_CBG, D), jnp.float32) for _ in range(4)],
            [pltpu.VMEM((_CBB, DQ), jnp.float32) for _ in range(4)],
            [pltpu.SemaphoreType.DMA for _ in range(4)],
            [pltpu.SemaphoreType.DMA for _ in range(4)],
            [pltpu.SemaphoreType.DMA for _ in range(4)],
        ],
    )
    def lookup(table_hbm, idx_hbm, out_hbm,
               idx_vg, idx_vb, tbl_v, gbuf, bbuf, sg, ssg, ssb):
        wid = lax.axis_index("s") * NC + lax.axis_index("c")
        gbase = wid * g_per_w
        span = wid // 4
        quarter = wid % 4
        bbase = G_TOTAL + span * b_per_s
        col = quarter * DQ
        pltpu.sync_copy(idx_hbm.at[pl.ds(gbase, g_per_w)], idx_vg)
        pltpu.sync_copy(idx_hbm.at[pl.ds(bbase, b_per_s)], idx_vb)
        pltpu.sync_copy(table_hbm.at[:, pl.ds(col, DQ)], tbl_v)

        def g_start(i, j):
            pltpu.async_copy(
                table_hbm.at[idx_vg.at[pl.ds(i * _CBG, _CBG)]], gbuf[j],
                sg[j])

        def g_wait(i, j):
            pltpu.make_async_copy(
                table_hbm.at[idx_vg.at[pl.ds(i * _CBG, _CBG)]], gbuf[j],
                sg[j]).wait()

        def g_out(i, j):
            pltpu.async_copy(
                gbuf[j], out_hbm.at[pl.ds(gbase + i * _CBG, _CBG)], ssg[j])

        def g_out_wait(i, j):
            pltpu.make_async_copy(
                gbuf[j], out_hbm.at[pl.ds(gbase + i * _CBG, _CBG)],
                ssg[j]).wait()

        def build(i, j):
            off = i * _CBB

            def grp(g, carry):
                vec = idx_vb[pl.ds(off + g * _L, _L)]
                rs = [vec[k] for k in range(_L)]

                @plsc.parallel_loop(0, DQ // _L, 1, unroll=DQ // _L)
                def col_body(jj):
                    for k in range(_L):
                        bbuf[j][g * _L + k, pl.ds(jj * _L, _L)] = (
                            tbl_v[rs[k], pl.ds(jj * _L, _L)])

                return carry

            lax.fori_loop(0, _CBB // _L, grp, 0)

        def b_out(i, j):
            pltpu.async_copy(
                bbuf[j],
                out_hbm.at[pl.ds(bbase + i * _CBB, _CBB), pl.ds(col, DQ)],
                ssb[j])

        def b_out_wait(i, j):
            pltpu.make_async_copy(
                bbuf[j],
                out_hbm.at[pl.ds(bbase + i * _CBB, _CBB), pl.ds(col, DQ)],
                ssb[j]).wait()

        def body(p, carry):
            for tt in range(4):
                ig = 4 * p + tt
                jg = tt

                @pl.when(ig >= 3)
                def _():
                    g_wait(ig - 3, (tt + 1) % 4)
                    g_out(ig - 3, (tt + 1) % 4)

                @pl.when(ig >= 4)
                def _():
                    g_out_wait(ig - 4, jg)

                g_start(ig, jg)

                for u in range(5):
                    ib = 20 * p + 5 * tt + u
                    jb = (5 * tt + u) % 4

                    @pl.when(ib >= 4)
                    def _():
                        b_out_wait(ib - 4, jb)

                    build(ib, jb)
                    b_out(ib, jb)
            return carry

        lax.fori_loop(0, NP, body, 0)
        for i in range(_NG - 3, _NG):
            g_wait(i, i % 4)
            g_out(i, i % 4)
        for i in range(_NG - 4, _NG):
            g_out_wait(i, i % 4)
        for i in range(_NB - 4, _NB):
            b_out_wait(i, i % 4)

    return lookup


def kernel(ids, kernel):
    rows, cols = ids.shape
    B = rows * cols
    idx = ids.reshape(B).astype(jnp.int32)
    out = _make_lookup(B, _DIM, kernel.shape[0])(kernel, idx)
    return out.reshape(rows, cols, _DIM)


# speedup vs baseline: 1.1719x; 1.1042x over previous
import functools

import jax
import jax.numpy as jnp
from jax import lax
from jax.experimental import pallas as pl
from jax.experimental.pallas import tpu as pltpu
from jax.experimental.pallas import tpu_sc as plsc

_DIM = 512
_NB = 2
_CB = 64


@functools.cache
def _make_lookup(B, D, V):
    info = plsc.get_sparse_core_info()
    NC, NS = info.num_cores, info.num_subcores
    NW = NC * NS
    DH = D // 2
    assert B % (NW // 2) == 0
    b_per_w = B // (NW // 2)
    NCH = b_per_w // _CB
    assert b_per_w % (_NB * _CB) == 0
    NP = NCH // _NB
    L = 16
    mesh = plsc.VectorSubcoreMesh(core_axis_name="c", subcore_axis_name="s")

    @functools.partial(
        pl.kernel,
        mesh=mesh,
        out_type=jax.ShapeDtypeStruct((B, D), jnp.float32),
        scratch_types=[
            pltpu.VMEM((b_per_w,), jnp.int32),
            pltpu.VMEM((V, DH), jnp.float32),
            [pltpu.VMEM((_CB, DH), jnp.float32) for _ in range(_NB)],
            [pltpu.SemaphoreType.DMA for _ in range(_NB)],
        ],
    )
    def lookup(table_hbm, idx_hbm, out_hbm, idx_v, tbl_v, rows, ss):
        wid = lax.axis_index("s") * NC + lax.axis_index("c")
        span = wid // 2
        half = wid % 2
        base = span * b_per_w
        col = half * DH
        pltpu.sync_copy(idx_hbm.at[pl.ds(base, b_per_w)], idx_v)
        pltpu.sync_copy(table_hbm.at[:, pl.ds(col, DH)], tbl_v)

        def build(c, jb):
            off = c * _CB

            @plsc.parallel_loop(0, _CB // L, 1, unroll=2)
            def grp_body(g):
                vec = idx_v[pl.ds(off + g * L, L)]
                rs = [vec[k] for k in range(L)]

                @plsc.parallel_loop(0, DH // L, 1, unroll=DH // L)
                def col_body(jj):
                    for k in range(L):
                        rows[jb][g * L + k, pl.ds(jj * L, L)] = (
                            tbl_v[rs[k], pl.ds(jj * L, L)])

        def scatter(c, jb):
            pltpu.async_copy(
                rows[jb],
                out_hbm.at[pl.ds(base + c * _CB, _CB), pl.ds(col, DH)],
                ss[jb])

        def scatter_wait(c, jb):
            pltpu.make_async_copy(
                rows[jb],
                out_hbm.at[pl.ds(base + c * _CB, _CB), pl.ds(col, DH)],
                ss[jb]).wait()

        def body(p, carry):
            for jb in range(_NB):
                c = _NB * p + jb

                @pl.when(c >= _NB)
                def _():
                    scatter_wait(c - _NB, jb)

                build(c, jb)
                scatter(c, jb)
            return carry

        lax.fori_loop(0, NP, body, 0)
        for jb in range(_NB):
            scatter_wait(NCH - _NB + jb, jb)

    return lookup


def kernel(ids, kernel):
    rows, cols = ids.shape
    B = rows * cols
    idx = ids.reshape(B).astype(jnp.int32)
    out = _make_lookup(B, _DIM, kernel.shape[0])(kernel, idx)
    return out.reshape(rows, cols, _DIM)


# speedup vs baseline: 1.9562x; 1.6693x over previous
import functools

import jax
import jax.numpy as jnp
from jax import lax
from jax.experimental import pallas as pl
from jax.experimental.pallas import tpu as pltpu
from jax.experimental.pallas import tpu_sc as plsc

_L = 16
_IDSR = 50
_LAP = 8
_NREG = 4
_SBL = 16


@functools.cache
def _make_lookup(R, C, D, V):
    info = plsc.get_sparse_core_info()
    NC, NS = info.num_cores, info.num_subcores
    NW = NC * NS
    DE = D // 4
    NSPAN = NW // 4
    rows_per_span = R // NSPAN
    NLAP = rows_per_span // _LAP
    NSB = NLAP // _SBL
    NP = NSB // 2
    FL = _LAP * _IDSR
    NG = FL // _L
    RR = _LAP // _NREG
    mesh = plsc.VectorSubcoreMesh(core_axis_name="c", subcore_axis_name="s")

    @functools.partial(
        pl.kernel,
        mesh=mesh,
        out_type=jax.ShapeDtypeStruct((R, C, D), jnp.float32),
        scratch_types=[
            pltpu.VMEM((V, DE), jnp.float32),
            pltpu.VMEM((_LAP, _IDSR, DE), jnp.float32),
            [pltpu.VMEM((_SBL * FL,), jnp.int32) for _ in range(2)],
            [pltpu.SemaphoreType.DMA for _ in range(_NREG)],
            [pltpu.SemaphoreType.DMA for _ in range(2)],
        ],
    )
    def lookup(table_hbm, idx_hbm, out_hbm, tbl_v, ring, idxb, sreg, sidx):
        wid = lax.axis_index("s") * NC + lax.axis_index("c")
        span = wid // 4
        col = (wid % 4) * DE
        qbase = span * rows_per_span
        fbase = qbase * _IDSR
        pltpu.sync_copy(table_hbm.at[:, pl.ds(col, DE)], tbl_v)

        def idx_load(sb, j):
            pltpu.async_copy(
                idx_hbm.at[pl.ds(fbase + sb * _SBL * FL, _SBL * FL)],
                idxb[j], sidx[j])

        def idx_wait(sb, j):
            pltpu.make_async_copy(
                idx_hbm.at[pl.ds(fbase + sb * _SBL * FL, _SBL * FL)],
                idxb[j], sidx[j]).wait()

        def reg_out(w, r):
            pltpu.async_copy(
                ring.at[pl.ds(r * RR, RR)],
                out_hbm.at[pl.ds(qbase + w * _LAP + r * RR, RR), :,
                           pl.ds(col, DE)],
                sreg[r])

        def reg_wait(w, r):
            pltpu.make_async_copy(
                ring.at[pl.ds(r * RR, RR)],
                out_hbm.at[pl.ds(qbase + w * _LAP + r * RR, RR), :,
                           pl.ds(col, DE)],
                sreg[r]).wait()

        idx_load(0, 0)
        idx_load(1, 1)

        def group(li, j, g):
            vec = idxb[j][pl.ds(li * FL + g * _L, _L)]
            rs = [vec[k] for k in range(_L)]
            if isinstance(g, int):
                qp = [divmod(g * _L + k, _IDSR) for k in range(_L)]
            else:
                qp = []
                for k in range(_L):
                    i = g * _L + k
                    q = i // _IDSR
                    qp.append((q, i - q * _IDSR))

            @plsc.parallel_loop(0, DE // _L, 1, unroll=DE // _L)
            def col_body(jj):
                for k in range(_L):
                    q, p = qp[k]
                    ring[q, p, pl.ds(jj * _L, _L)] = (
                        tbl_v[rs[k], pl.ds(jj * _L, _L)])

        def lap(w, li, j):
            def seg(lo, hi):
                def b(g, c):
                    group(li, j, g)
                    return c

                lax.fori_loop(lo, hi, b, 0)

            def wait_reg(r):
                @pl.when(w >= 1)
                def _():
                    reg_wait(w - 1, r)

            wait_reg(0)
            seg(0, 6)
            wait_reg(1)
            group(li, j, 6)
            reg_out(w, 0)
            seg(7, 12)
            wait_reg(2)
            group(li, j, 12)
            reg_out(w, 1)
            seg(13, 18)
            wait_reg(3)
            group(li, j, 18)
            reg_out(w, 2)
            seg(19, 24)
            group(li, j, 24)
            reg_out(w, 3)

        def body(p, carry):
            for sbl in range(2):
                sb = 2 * p + sbl
                idx_wait(sb, sbl)

                def inner(li, c2):
                    lap(sb * _SBL + li, li, sbl)
                    return c2

                lax.fori_loop(0, _SBL, inner, 0)

                @pl.when(sb < NSB - 2)
                def _(sbl=sbl):
                    idx_load(sb + 2, sbl)
            return carry

        lax.fori_loop(0, NP, body, 0)
        for r in range(_NREG):
            reg_wait(NLAP - 1, r)

    return lookup


def kernel(ids, kernel):
    rows, cols = ids.shape
    idx = ids.reshape(rows * cols).astype(jnp.int32)
    return _make_lookup(rows, cols, kernel.shape[1], kernel.shape[0])(
        kernel, idx)
